# baseline (speedup 1.0000x reference)
import jax
import jax.numpy as jnp
from jax.experimental import pallas as pl

_MB = 10


def _embed_kernel(x_ref, posb_ref, wt_ref, o_ref):
    wt = wt_ref[...]
    base = pl.program_id(0) * _MB
    for t in range(x_ref.shape[0]):
        acc = jax.lax.dot_general(
            wt, x_ref[t], (((1,), (0,)), ((), ())),
            preferred_element_type=jnp.float32)
        o_ref[t] = acc + posb_ref[base + t][:, None]


def kernel(x, pos_table, W, b):
    B, L, D = x.shape
    xt = jnp.transpose(x, (1, 2, 0))
    posb = pos_table + b[None, :]
    wt = W.T

    out = pl.pallas_call(
        _embed_kernel,
        grid=(L // _MB,),
        in_specs=[
            pl.BlockSpec((_MB, D, B), lambda i: (i, 0, 0)),
            pl.BlockSpec((L, D), lambda i: (0, 0)),
            pl.BlockSpec((D, D), lambda i: (0, 0)),
        ],
        out_specs=pl.BlockSpec((_MB, D, B), lambda i: (i, 0, 0)),
        out_shape=jax.ShapeDtypeStruct((L, D, B), x.dtype),
    )(xt, posb, wt)
    return jnp.transpose(out, (2, 0, 1))


# speedup vs baseline: 1.0364x; 1.0364x over previous
import jax
import jax.numpy as jnp
from jax.experimental import pallas as pl

_MB = 25


def _embed_kernel(x_ref, posb_ref, wt_ref, o_ref):
    wt = wt_ref[...]
    base = pl.program_id(0) * _MB
    for t in range(x_ref.shape[0]):
        acc = jax.lax.dot_general(
            wt, x_ref[t], (((1,), (0,)), ((), ())),
            preferred_element_type=jnp.float32)
        o_ref[t] = acc + posb_ref[base + t][:, None]


def kernel(x, pos_table, W, b):
    B, L, D = x.shape
    xt = jnp.transpose(x, (1, 2, 0))
    posb = pos_table + b[None, :]
    wt = W.T

    out = pl.pallas_call(
        _embed_kernel,
        grid=(L // _MB,),
        in_specs=[
            pl.BlockSpec((_MB, D, B), lambda i: (i, 0, 0)),
            pl.BlockSpec((L, D), lambda i: (0, 0)),
            pl.BlockSpec((D, D), lambda i: (0, 0)),
        ],
        out_specs=pl.BlockSpec((_MB, D, B), lambda i: (i, 0, 0)),
        out_shape=jax.ShapeDtypeStruct((L, D, B), x.dtype),
    )(xt, posb, wt)
    return jnp.transpose(out, (2, 0, 1))


# speedup vs baseline: 1.0611x; 1.0238x over previous
import jax
import jax.numpy as jnp
from jax.experimental import pallas as pl

_MB = 16


def _embed_kernel(x_ref, posb_ref, wt_ref, o_ref):
    wt = wt_ref[...]
    base = pl.program_id(0) * _MB
    for t in range(x_ref.shape[0]):
        acc = jax.lax.dot_general(
            wt, x_ref[t], (((1,), (0,)), ((), ())),
            preferred_element_type=jnp.float32)
        o_ref[t] = acc + posb_ref[base + t][:, None]


def kernel(x, pos_table, W, b):
    B, L, D = x.shape
    xt = jnp.transpose(x, (1, 2, 0))
    posb = pos_table + b[None, :]
    wt = W.T

    out = pl.pallas_call(
        _embed_kernel,
        grid=(L // _MB,),
        in_specs=[
            pl.BlockSpec((_MB, D, B), lambda i: (i, 0, 0)),
            pl.BlockSpec((L, D), lambda i: (0, 0)),
            pl.BlockSpec((D, D), lambda i: (0, 0)),
        ],
        out_specs=pl.BlockSpec((_MB, D, B), lambda i: (i, 0, 0)),
        out_shape=jax.ShapeDtypeStruct((L, D, B), x.dtype),
    )(xt, posb, wt)
    return jnp.transpose(out, (2, 0, 1))
